# Initial kernel scaffold; baseline (speedup 1.0000x reference)
import jax
import jax.numpy as jnp
from jax.experimental import pallas as pl


def kernel(edge_attr, emb0, emb1, emb2):
    raise NotImplementedError("write your pallas kernel here")


# speedup vs baseline: 1.4851x; 1.4851x over previous
import jax
import jax.numpy as jnp
from jax import lax
from jax.experimental import pallas as pl
from jax.experimental.pallas import tpu as pltpu
from jax.experimental.pallas import tpu_sc as plsc

N_EDGES = 1600000
D = 32
N0, N1, N2 = 22, 6, 2
NT = N0 * N1 * N2
NC, NS, L = 2, 16, 16
NW = NC * NS
PER_W = N_EDGES // NW
CHUNK = 2000
N_CHUNKS = PER_W // CHUNK
GROUPS = CHUNK // L


def _body(attr_hbm, emb0_hbm, emb1_hbm, emb2_hbm, out_hbm,
          attr_v, out_v, t_v, e0_v, e1_v, e2_v):
    wid = lax.axis_index("s") * NC + lax.axis_index("c")

    pltpu.sync_copy(emb0_hbm, e0_v)
    pltpu.sync_copy(emb1_hbm, e1_v)
    pltpu.sync_copy(emb2_hbm, e2_v)

    def build_row(j, carry):
        a0 = j // (N1 * N2)
        rem = j - a0 * (N1 * N2)
        a1 = rem // N2
        a2 = rem - a1 * N2
        for h in (0, 16):
            t_v[pl.ds(j * D + h, 16)] = (
                e0_v[pl.ds(a0 * D + h, 16)]
                + e1_v[pl.ds(a1 * D + h, 16)]
                + e2_v[pl.ds(a2 * D + h, 16)]
            )
        return carry

    lax.fori_loop(0, NT, build_row, 0)

    iota = lax.iota(jnp.int32, L)

    def do_chunk(g, carry):
        base = wid * PER_W + g * CHUNK
        pltpu.sync_copy(attr_hbm.at[pl.ds(base * 3, CHUNK * 3)], attr_v)

        def do_group(i, c2):
            rows3 = (i * L + iota) * 3
            a0 = plsc.load_gather(attr_v, [rows3])
            a1 = plsc.load_gather(attr_v, [rows3 + 1])
            a2 = plsc.load_gather(attr_v, [rows3 + 2])
            rD = (a0 * (N1 * N2) + a1 * N2 + a2) * D
            outbase = (i * L + iota) * D
            for c in range(D):
                v = plsc.load_gather(t_v, [rD + c])
                plsc.store_scatter(out_v, [outbase + c], v)
            return c2

        lax.fori_loop(0, GROUPS, do_group, 0)
        pltpu.sync_copy(out_v, out_hbm.at[pl.ds(base * D, CHUNK * D)])
        return carry

    lax.fori_loop(0, N_CHUNKS, do_chunk, 0)


@jax.jit
def kernel(edge_attr, emb0, emb1, emb2):
    mesh = plsc.VectorSubcoreMesh(core_axis_name="c", subcore_axis_name="s")
    k = pl.kernel(
        _body,
        out_type=jax.ShapeDtypeStruct((N_EDGES * D,), jnp.float32),
        mesh=mesh,
        compiler_params=pltpu.CompilerParams(needs_layout_passes=False),
        scratch_types=[
            pltpu.VMEM((CHUNK * 3,), jnp.int32),
            pltpu.VMEM((CHUNK * D,), jnp.float32),
            pltpu.VMEM((NT * D,), jnp.float32),
            pltpu.VMEM((N0 * D,), jnp.float32),
            pltpu.VMEM((N1 * D,), jnp.float32),
            pltpu.VMEM((N2 * D,), jnp.float32),
        ],
    )
    out = k(edge_attr.reshape(-1), emb0.reshape(-1), emb1.reshape(-1),
            emb2.reshape(-1))
    return out.reshape(N_EDGES, D)
